# baseline (speedup 1.0000x reference)
import functools

import jax
import jax.numpy as jnp
from jax import lax
from jax.experimental import pallas as pl
from jax.experimental.pallas import tpu as pltpu
from jax.experimental.pallas import tpu_sc as plsc

BATCH = 4096
FEAT = 128
NCLS_PAD = 100352
ROW_TILE = 2048
CHUNK = 128


def _make_sc_gather_counts():
    info = plsc.get_sparse_core_info()
    nc, ns = 1, info.num_subcores
    nw = nc * ns
    b_per_w = BATCH // nw
    n_chunks = BATCH // CHUNK
    mesh = plsc.VectorSubcoreMesh(
        core_axis_name="c", subcore_axis_name="s", num_cores=1)

    @functools.partial(
        pl.kernel,
        mesh=mesh,
        out_type=(
            jax.ShapeDtypeStruct((BATCH, FEAT), jnp.float32),
            jax.ShapeDtypeStruct((BATCH,), jnp.float32),
        ),
        scratch_types=[
            pltpu.VMEM((2 * CHUNK, FEAT), jnp.float32),
            pltpu.VMEM((n_chunks, CHUNK), jnp.int32),
            pltpu.VMEM((CHUNK,), jnp.int32),
            pltpu.VMEM((CHUNK,), jnp.int32),
            pltpu.VMEM((CHUNK,), jnp.int32),
            pltpu.VMEM((CHUNK,), jnp.int32),
            pltpu.VMEM((CHUNK,), jnp.int32),
            pltpu.VMEM((CHUNK,), jnp.int32),
            pltpu.VMEM((2, CHUNK), jnp.float32),
            pltpu.VMEM_SHARED((NCLS_PAD,), jnp.int32),
            pltpu.SemaphoreType.DMA,
            pltpu.SemaphoreType.DMA,
        ],
    )
    def gather_counts(centers_hbm, labels2d_hbm, bc_hbm, cnt_hbm,
                      rows_v, lab2d, ones_v, cidx_a, cidx_b,
                      pre_a, pre_b, post_v, cnt_v, table, sem, sem2):
        sid = lax.axis_index("s")
        pltpu.sync_copy(labels2d_hbm.at[2 * sid], cidx_a)
        pltpu.sync_copy(labels2d_hbm.at[2 * sid + 1], cidx_b)
        gat_a = pltpu.async_copy(
            centers_hbm.at[cidx_a], rows_v.at[pl.ds(0, CHUNK)], sem)
        gat_b = pltpu.async_copy(
            centers_hbm.at[cidx_b], rows_v.at[pl.ds(CHUNK, CHUNK)], sem)

        pltpu.async_copy(table.at[cidx_a], pre_a, sem2).wait()
        pltpu.async_copy(table.at[cidx_b], pre_b, sem2).wait()
        plsc.subcore_barrier()
        @pl.when(sid == 0)
        def _():
            pltpu.sync_copy(labels2d_hbm, lab2d)
            for i in range(CHUNK // 16):
                ones_v[pl.ds(i * 16, 16)] = jnp.ones((16,), jnp.int32)
            for j in range(n_chunks):
                pltpu.sync_copy(ones_v, table.at[lab2d.at[j]], add=True)
        plsc.subcore_barrier()
        pltpu.async_copy(table.at[cidx_a], post_v, sem2).wait()
        for i in range(CHUNK // 16):
            sl = pl.ds(i * 16, 16)
            cnt_v[0, sl] = (post_v[sl] - pre_a[sl]).astype(jnp.float32)
        pltpu.async_copy(table.at[cidx_b], post_v, sem2).wait()
        for i in range(CHUNK // 16):
            sl = pl.ds(i * 16, 16)
            cnt_v[1, sl] = (post_v[sl] - pre_b[sl]).astype(jnp.float32)
        pltpu.sync_copy(cnt_v.at[0], cnt_hbm.at[pl.ds(2 * sid * CHUNK, CHUNK)])
        pltpu.sync_copy(cnt_v.at[1], cnt_hbm.at[pl.ds((2 * sid + 1) * CHUNK, CHUNK)])

        gat_a.wait()
        gat_b.wait()
        pltpu.sync_copy(rows_v, bc_hbm.at[pl.ds(2 * sid * CHUNK, 2 * CHUNK)])

    return gather_counts


_sc_gather_counts = _make_sc_gather_counts()


def _loss_body(x_ref, bc_ref, c_ref, vec_ref, out_ref):
    i = pl.program_id(0)
    x = x_ref[...]
    bc = bc_ref[...]
    x16 = x.astype(jnp.bfloat16)
    bc16 = bc.astype(jnp.bfloat16)
    s = lax.dot_general(
        x16, bc16, (((1,), (1,)), ((), ())),
        preferred_element_type=jnp.float32,
    )
    r = jnp.maximum(s, 0.0)
    rowsum = jnp.sum(r * r, axis=0, keepdims=True)
    bcrow = bc_ref[pl.ds(i * ROW_TILE, ROW_TILE), :]
    p = x * bcrow
    ones = jnp.ones((FEAT, FEAT), jnp.float32)
    q2d = lax.dot_general(
        p, ones, (((1,), (0,)), ((), ())),
        preferred_element_type=jnp.float32,
    )
    c = c_ref[0, :]
    corr = c[:, None] * (1.0 - 2.0 * jnp.maximum(q2d, 0.0))
    csum = jnp.sum(corr, axis=0, keepdims=True)

    @pl.when(i == 0)
    def _():
        vec_ref[...] = jnp.zeros_like(vec_ref)
        out_ref[0, 0] = 0.0

    vec_ref[...] += rowsum
    out_ref[0, 0] += jnp.sum(csum) * (1.0 / FEAT)

    @pl.when(i == pl.num_programs(0) - 1)
    def _():
        out_ref[0, 0] += jnp.sum(vec_ref[...])


def _fused_loss(x, bc, counts2d):
    grid = (BATCH // ROW_TILE,)
    return pl.pallas_call(
        _loss_body,
        grid=grid,
        in_specs=[
            pl.BlockSpec((ROW_TILE, FEAT), lambda i: (i, 0)),
            pl.BlockSpec((BATCH, FEAT), lambda i: (0, 0)),
            pl.BlockSpec((1, ROW_TILE), lambda i: (0, i)),
        ],
        out_specs=[
            pl.BlockSpec((1, BATCH), lambda i: (0, 0)),
            pl.BlockSpec((1, 1), lambda i: (0, 0), memory_space=pltpu.SMEM),
        ],
        out_shape=[
            jax.ShapeDtypeStruct((1, BATCH), jnp.float32),
            jax.ShapeDtypeStruct((1, 1), jnp.float32),
        ],
    )(x, bc, counts2d)


@jax.jit
def kernel(x, labels, centers):
    labels2d = labels.reshape(BATCH // CHUNK, CHUNK)
    batch_centers, counts = _sc_gather_counts(centers, labels2d)
    _, out = _fused_loss(x, batch_centers, counts.reshape(1, BATCH))
    return out[0, 0]


# speedup vs baseline: 1.0185x; 1.0185x over previous
import functools

import jax
import jax.numpy as jnp
from jax import lax
from jax.experimental import pallas as pl
from jax.experimental.pallas import tpu as pltpu
from jax.experimental.pallas import tpu_sc as plsc

BATCH = 4096
FEAT = 128
NCLS_PAD = 100352
ROW_TILE = 1024
CHUNK = 128


def _make_sc_gather_counts():
    info = plsc.get_sparse_core_info()
    nc, ns = info.num_cores, info.num_subcores
    nw = nc * ns
    b_per_w = BATCH // nw
    n_chunks = BATCH // CHUNK
    assert b_per_w == CHUNK
    mesh = plsc.VectorSubcoreMesh(core_axis_name="c", subcore_axis_name="s")

    @functools.partial(
        pl.kernel,
        mesh=mesh,
        out_type=(
            jax.ShapeDtypeStruct((BATCH, FEAT), jnp.float32),
            jax.ShapeDtypeStruct((2, BATCH), jnp.float32),
        ),
        scratch_types=[
            pltpu.VMEM((CHUNK,), jnp.int32),
            pltpu.VMEM((CHUNK, FEAT), jnp.float32),
            pltpu.VMEM((n_chunks // 2, CHUNK), jnp.int32),
            pltpu.VMEM((CHUNK,), jnp.int32),
            pltpu.VMEM((CHUNK,), jnp.int32),
            pltpu.VMEM((CHUNK,), jnp.int32),
            pltpu.VMEM((CHUNK,), jnp.int32),
            pltpu.VMEM((CHUNK,), jnp.int32),
            pltpu.VMEM((CHUNK,), jnp.int32),
            pltpu.VMEM((2, CHUNK), jnp.float32),
            pltpu.VMEM_SHARED((NCLS_PAD,), jnp.int32),
            pltpu.SemaphoreType.DMA,
            pltpu.SemaphoreType.DMA,
        ],
    )
    def gather_counts(centers_hbm, labels2d_hbm, bc_hbm, cnt_hbm,
                      idx_v, rows_v, lab2d, ones_v, cidx_a, cidx_b,
                      pre_a, pre_b, post_v, cnt_v, table, sem, sem2):
        cid = lax.axis_index("c")
        sid = lax.axis_index("s")
        wid = sid * nc + cid
        pltpu.sync_copy(labels2d_hbm.at[wid], idx_v)
        gat = pltpu.async_copy(centers_hbm.at[idx_v], rows_v, sem)

        pltpu.sync_copy(labels2d_hbm.at[2 * sid], cidx_a)
        pltpu.sync_copy(labels2d_hbm.at[2 * sid + 1], cidx_b)
        pltpu.async_copy(table.at[cidx_a], pre_a, sem2).wait()
        pltpu.async_copy(table.at[cidx_b], pre_b, sem2).wait()
        plsc.subcore_barrier()
        @pl.when(sid == 0)
        def _():
            pltpu.sync_copy(
                labels2d_hbm.at[pl.ds(cid * (n_chunks // 2), n_chunks // 2)],
                lab2d)
            for i in range(CHUNK // 16):
                ones_v[pl.ds(i * 16, 16)] = jnp.ones((16,), jnp.int32)
            for j in range(n_chunks // 2):
                pltpu.sync_copy(ones_v, table.at[lab2d.at[j]], add=True)
        plsc.subcore_barrier()
        pltpu.async_copy(table.at[cidx_a], post_v, sem2).wait()
        for i in range(CHUNK // 16):
            sl = pl.ds(i * 16, 16)
            cnt_v[0, sl] = (post_v[sl] - pre_a[sl]).astype(jnp.float32)
        pltpu.async_copy(table.at[cidx_b], post_v, sem2).wait()
        for i in range(CHUNK // 16):
            sl = pl.ds(i * 16, 16)
            cnt_v[1, sl] = (post_v[sl] - pre_b[sl]).astype(jnp.float32)
        pltpu.sync_copy(cnt_v.at[0], cnt_hbm.at[cid, pl.ds(2 * sid * CHUNK, CHUNK)])
        pltpu.sync_copy(cnt_v.at[1], cnt_hbm.at[cid, pl.ds((2 * sid + 1) * CHUNK, CHUNK)])

        gat.wait()
        pltpu.sync_copy(rows_v, bc_hbm.at[pl.ds(wid * CHUNK, CHUNK)])

    return gather_counts


_sc_gather_counts = _make_sc_gather_counts()


def _loss_body(x_ref, bc_ref, c_ref, vec_ref, out_ref):
    i = pl.program_id(0)
    x = x_ref[...]
    bc = bc_ref[...]
    x16 = x.astype(jnp.bfloat16)
    bc16 = bc.astype(jnp.bfloat16)
    s = lax.dot_general(
        x16, bc16, (((1,), (1,)), ((), ())),
        preferred_element_type=jnp.float32,
    )
    r = jnp.maximum(s, 0.0)
    rowsum = jnp.sum(r * r, axis=0, keepdims=True)
    bcrow = bc_ref[pl.ds(i * ROW_TILE, ROW_TILE), :]
    p = x * bcrow
    ones = jnp.ones((FEAT, FEAT), jnp.float32)
    q2d = lax.dot_general(
        p, ones, (((1,), (0,)), ((), ())),
        preferred_element_type=jnp.float32,
    )
    c = c_ref[0, :] + c_ref[1, :]
    corr = c[:, None] * (1.0 - 2.0 * jnp.maximum(q2d, 0.0))
    csum = jnp.sum(corr, axis=0, keepdims=True)

    @pl.when(i == 0)
    def _():
        vec_ref[...] = jnp.zeros_like(vec_ref)
        out_ref[0, 0] = 0.0

    vec_ref[...] += rowsum
    out_ref[0, 0] += jnp.sum(csum) * (1.0 / FEAT)

    @pl.when(i == pl.num_programs(0) - 1)
    def _():
        out_ref[0, 0] += jnp.sum(vec_ref[...])


def _fused_loss(x, bc, counts2d):
    grid = (BATCH // ROW_TILE,)
    return pl.pallas_call(
        _loss_body,
        grid=grid,
        in_specs=[
            pl.BlockSpec((ROW_TILE, FEAT), lambda i: (i, 0)),
            pl.BlockSpec((BATCH, FEAT), lambda i: (0, 0)),
            pl.BlockSpec((2, ROW_TILE), lambda i: (0, i)),
        ],
        out_specs=[
            pl.BlockSpec((1, BATCH), lambda i: (0, 0)),
            pl.BlockSpec((1, 1), lambda i: (0, 0), memory_space=pltpu.SMEM),
        ],
        out_shape=[
            jax.ShapeDtypeStruct((1, BATCH), jnp.float32),
            jax.ShapeDtypeStruct((1, 1), jnp.float32),
        ],
    )(x, bc, counts2d)


@jax.jit
def kernel(x, labels, centers):
    labels2d = labels.reshape(BATCH // CHUNK, CHUNK)
    batch_centers, counts = _sc_gather_counts(centers, labels2d)
    _, out = _fused_loss(x, batch_centers, counts)
    return out[0, 0]
